# baseline (speedup 1.0000x reference)
import functools

import jax
import jax.numpy as jnp
from jax import lax
from jax.experimental import pallas as pl
from jax.experimental.pallas import tpu as pltpu
from jax.experimental.pallas import tpu_sc as plsc

B = 16384
F = 26
V = 100000
D = 13
E = 16
FE = F * E

NC = 2
NS = 16
NW = NC * NS

NIDX = (B * F) // NW
CW = 128
NCH = NIDX // CW
K = 8
NSTEP = NCH // K


def _sc_gather_body(idx_hbm, tfm_hbm, tlin_hbm, fm_hbm, lin_hbm,
                    idx_v, rows_v, lin_v, sem, sem2):
    c = lax.axis_index("c")
    s = lax.axis_index("s")
    wid = s * NC + c
    pltpu.sync_copy(idx_hbm.at[wid], idx_v)

    def step(j, carry):
        cp1 = pltpu.async_copy(tfm_hbm.at[idx_v.at[j]], rows_v, sem)
        cp2 = pltpu.async_copy(tlin_hbm.at[idx_v.at[j]], lin_v, sem2)
        cp1.wait()
        cp2.wait()
        pltpu.sync_copy(rows_v, fm_hbm.at[wid, j])
        pltpu.sync_copy(lin_v, lin_hbm.at[wid, j])
        return carry

    lax.fori_loop(0, NCH, step, 0)


@functools.cache
def _sc_gather_kernel():
    return pl.kernel(
        _sc_gather_body,
        out_type=(
            jax.ShapeDtypeStruct((NW, NCH, CW, E), jnp.float32),
            jax.ShapeDtypeStruct((NW, NCH, CW), jnp.float32),
        ),
        mesh=plsc.VectorSubcoreMesh(core_axis_name="c", subcore_axis_name="s",
                                    num_cores=NC, num_subcores=NS),
        scratch_types=[
            pltpu.VMEM((NCH, CW), jnp.int32),
            pltpu.VMEM((CW, E), jnp.float32),
            pltpu.VMEM((CW,), jnp.float32),
            pltpu.SemaphoreType.DMA,
            pltpu.SemaphoreType.DMA,
        ],
        compiler_params=pltpu.CompilerParams(use_tc_tiling_on_sc=False),
    )


def _tc_body(dense_ref, fm_ref, lin_ref, Wd_ref, W1a_ref, W1b_ref, b1_ref,
             W2_ref, b2_ref, W3_ref, b3_ref, W4_ref, sc_ref,
             fin_ref, like_ref):
    fm = fm_ref[...]
    dense = dense_ref[...]
    linv = lin_ref[...]

    j = lax.broadcasted_iota(jnp.int32, (FE, E), 0)
    e = lax.broadcasted_iota(jnp.int32, (FE, E), 1)
    S = (j % E == e).astype(jnp.float32)
    summed = jnp.dot(fm, S, preferred_element_type=jnp.float32)
    sqsum = jnp.dot(fm * fm, S, preferred_element_type=jnp.float32)
    second = 0.5 * jnp.sum(summed * summed - sqsum, axis=1, keepdims=True)

    first = (jnp.dot(dense, Wd_ref[...], preferred_element_type=jnp.float32)
             + jnp.sum(linv, axis=1, keepdims=True) + sc_ref[0])

    h = jnp.maximum(
        jnp.dot(dense, W1a_ref[...], preferred_element_type=jnp.float32)
        + jnp.dot(fm, W1b_ref[...], preferred_element_type=jnp.float32)
        + b1_ref[...], 0.0)
    h = jnp.maximum(
        jnp.dot(h, W2_ref[...], preferred_element_type=jnp.float32)
        + b2_ref[...], 0.0)
    h = jnp.maximum(
        jnp.dot(h, W3_ref[...], preferred_element_type=jnp.float32)
        + b3_ref[...], 0.0)
    dnn = jnp.dot(h, W4_ref[...], preferred_element_type=jnp.float32) + sc_ref[1]

    logits = first + second + dnn
    fin_ref[...] = jax.nn.sigmoid(logits * sc_ref[2] + sc_ref[3])
    like_ref[...] = jax.nn.sigmoid(logits * sc_ref[4] + sc_ref[5])


BLK = 512


def _tc_call(dense, fm_flat, lin, Wd, W1a, W1b, b1, W2, b2, W3, b3, W4, scal):
    nblk = B // BLK
    full = lambda shape: pl.BlockSpec(shape, lambda i: (0,) * len(shape))
    row = lambda width: pl.BlockSpec((BLK, width), lambda i: (i, 0))
    return pl.pallas_call(
        _tc_body,
        grid=(nblk,),
        in_specs=[
            row(D), row(FE), row(F),
            full((D, 1)), full((D, 200)), full((FE, 200)), full((1, 200)),
            full((200, 200)), full((1, 200)),
            full((200, 200)), full((1, 200)),
            full((200, 1)),
            pl.BlockSpec(memory_space=pltpu.SMEM),
        ],
        out_specs=[row(1), row(1)],
        out_shape=[
            jax.ShapeDtypeStruct((B, 1), jnp.float32),
            jax.ShapeDtypeStruct((B, 1), jnp.float32),
        ],
    )(dense, fm_flat, lin, Wd, W1a, W1b, b1, W2, b2, W3, b3, W4, scal)


def kernel(sparse_inputs, dense_inputs, Wd, bd, Tlin, Tfm,
           W1, b1, W2, b2, W3, b3, W4, b4, Wf, bf, Wl, bl):
    flat_idx = (sparse_inputs
                + jnp.arange(F, dtype=jnp.int32)[None, :] * V)
    flat_idx = flat_idx.reshape(NW, NCH, CW)
    tfm_flat = Tfm.reshape(F * V, E)
    tlin_flat = Tlin.reshape(F * V)

    fm4, lin3 = _sc_gather_kernel()(flat_idx, tfm_flat, tlin_flat)
    fm_flat = fm4.reshape(B, FE)
    lin = lin3.reshape(B, F)

    scal = jnp.concatenate([
        bd, b4, Wf[0], bf, Wl[0], bl, jnp.zeros((2,), jnp.float32)])
    fin, like = _tc_call(dense_inputs, fm_flat, lin,
                         Wd, W1[:D], W1[D:], b1.reshape(1, 200),
                         W2, b2.reshape(1, 200), W3, b3.reshape(1, 200),
                         W4, scal)
    return (fin, like)
